# Initial kernel scaffold; baseline (speedup 1.0000x reference)
import jax
import jax.numpy as jnp
from jax.experimental import pallas as pl


def kernel(x, regime, ln_gamma, ln_beta, W1, b1, W2, b2, Wr1, br1, Wr2, br2):
    raise NotImplementedError("write your pallas kernel here")


# speedup vs baseline: 1.6429x; 1.6429x over previous
import functools

import jax
import jax.numpy as jnp
from jax.experimental import pallas as pl
from jax.experimental.pallas import tpu as pltpu

B, T, D = 1, 2048, 768
H, E, K, R = 1024, 8, 2, 5
LBW = 0.01

BT = 512
NT = T // BT


def _moe_kernel(x_ref, regime_ref, gamma_ref, beta_ref,
                w1_ref, b1_ref, w2_ref, b2_ref,
                wr1a_ref, wr1b_ref, br1_ref, wr2_ref, br2_ref,
                out_ref, aux_ref,
                xn_bf, wfull, aux_acc, acc):
    e = pl.program_id(0)
    t = pl.program_id(1)
    xblk = x_ref[...]

    @pl.when(e == 0)
    def _router():
        mean = jnp.mean(xblk, axis=1, keepdims=True)
        xc = xblk - mean
        var = jnp.mean(xc * xc, axis=1, keepdims=True)
        xn = xc * jax.lax.rsqrt(var + 1e-5) * gamma_ref[...] + beta_ref[...]
        xn_bf[pl.ds(t * BT, BT), :] = xn.astype(jnp.bfloat16)
        rc = jnp.dot(regime_ref[...], wr1b_ref[...],
                     preferred_element_type=jnp.float32)
        hpre = (jnp.dot(xn, wr1a_ref[...], preferred_element_type=jnp.float32)
                + rc + br1_ref[...])
        hrt = hpre * jax.nn.sigmoid(hpre)
        logits = (jnp.dot(hrt, wr2_ref[...], preferred_element_type=jnp.float32)
                  + br2_ref[...])
        ecols = jax.lax.broadcasted_iota(jnp.int32, (BT, E), 1)
        m1 = jnp.max(logits, axis=1, keepdims=True)
        i1 = jnp.min(jnp.where(logits == m1, ecols, E), axis=1, keepdims=True)
        masked = jnp.where(ecols == i1, -jnp.inf, logits)
        m2 = jnp.max(masked, axis=1, keepdims=True)
        i2 = jnp.min(jnp.where(masked == m2, ecols, E), axis=1, keepdims=True)
        w_first = 1.0 / (1.0 + jnp.exp(m2 - m1))
        wsel = (jnp.where(ecols == i1, w_first, 0.0)
                + jnp.where(ecols == i2, 1.0 - w_first, 0.0))
        wfull[pl.ds(t * BT, BT), :] = wsel
        p = jnp.exp(logits - m1)
        p = p / jnp.sum(p, axis=1, keepdims=True)
        pa = jnp.sum(p, axis=0, keepdims=True) / T
        ma = jnp.sum(jnp.where(ecols == i1, 1.0, 0.0), axis=0,
                     keepdims=True) / T

        @pl.when(t == 0)
        def _():
            aux_acc[0:1, 0:E] = pa
            aux_acc[1:2, 0:E] = ma

        @pl.when(t > 0)
        def _():
            aux_acc[0:1, 0:E] += pa
            aux_acc[1:2, 0:E] += ma

        @pl.when(t == NT - 1)
        def _():
            aux_ref[...] = (LBW * E) * jnp.sum(
                aux_acc[0:1, 0:E] * aux_acc[1:2, 0:E], axis=1,
                keepdims=True)

    ecols = jax.lax.broadcasted_iota(jnp.int32, (BT, E), 1)
    w_col = jnp.sum(wfull[pl.ds(t * BT, BT), :] * (ecols == e),
                    axis=1, keepdims=True)
    xb = xn_bf[pl.ds(t * BT, BT), :]
    h = jnp.dot(xb, w1_ref[0], preferred_element_type=jnp.float32) + b1_ref[0]
    h = h * jax.nn.sigmoid(h)
    y = (jnp.dot(h.astype(jnp.bfloat16), w2_ref[0],
                 preferred_element_type=jnp.float32) + b2_ref[0])
    contrib = w_col * y

    @pl.when(e == 0)
    def _():
        acc[pl.ds(t * BT, BT), :] = xblk + contrib

    @pl.when(e > 0)
    def _():
        acc[pl.ds(t * BT, BT), :] += contrib

    @pl.when(e == E - 1)
    def _():
        out_ref[...] = acc[pl.ds(t * BT, BT), :]


def _run(x2d, regime, gamma, beta, w1b, b1, w2b, b2, wr1a, wr1b, br1, wr2, br2):
    return pl.pallas_call(
        _moe_kernel,
        grid=(E, NT),
        in_specs=[
            pl.BlockSpec((BT, D), lambda e, t: (t, 0)),
            pl.BlockSpec((B, R), lambda e, t: (0, 0)),
            pl.BlockSpec((1, D), lambda e, t: (0, 0)),
            pl.BlockSpec((1, D), lambda e, t: (0, 0)),
            pl.BlockSpec((1, D, H), lambda e, t: (e, 0, 0)),
            pl.BlockSpec((1, 1, H), lambda e, t: (e, 0, 0)),
            pl.BlockSpec((1, H, D), lambda e, t: (e, 0, 0)),
            pl.BlockSpec((1, 1, D), lambda e, t: (e, 0, 0)),
            pl.BlockSpec((D, D), lambda e, t: (0, 0)),
            pl.BlockSpec((R, D), lambda e, t: (0, 0)),
            pl.BlockSpec((1, D), lambda e, t: (0, 0)),
            pl.BlockSpec((D, E), lambda e, t: (0, 0)),
            pl.BlockSpec((1, E), lambda e, t: (0, 0)),
        ],
        out_specs=[
            pl.BlockSpec((BT, D), lambda e, t: (t, 0)),
            pl.BlockSpec((1, 1), lambda e, t: (0, 0)),
        ],
        out_shape=[
            jax.ShapeDtypeStruct((T, D), jnp.float32),
            jax.ShapeDtypeStruct((1, 1), jnp.float32),
        ],
        scratch_shapes=[
            pltpu.VMEM((T, D), jnp.bfloat16),
            pltpu.VMEM((T, E), jnp.float32),
            pltpu.VMEM((8, 128), jnp.float32),
            pltpu.VMEM((T, D), jnp.float32),
        ],
    )(x2d, regime, gamma, beta, w1b, b1, w2b, b2, wr1a, wr1b, br1, wr2, br2)


def kernel(x, regime, ln_gamma, ln_beta, W1, b1, W2, b2, Wr1, br1, Wr2, br2):
    x2d = x.reshape(T, D)
    out2d, aux = _run(
        x2d, regime, ln_gamma.reshape(1, D), ln_beta.reshape(1, D),
        W1.astype(jnp.bfloat16), b1.reshape(E, 1, H),
        W2.astype(jnp.bfloat16), b2.reshape(E, 1, D),
        Wr1[:D], Wr1[D:], br1.reshape(1, D), Wr2, br2.reshape(1, E))
    return out2d.reshape(B, T, D), aux[0, 0]
